# baseline (speedup 1.0000x reference)
import jax
import jax.numpy as jnp
from jax.experimental import pallas as pl
from jax.experimental.pallas import tpu as pltpu

_BM = 400


def _h0_body(x_ref, w1_ref, o_ref):
    o_ref[...] = jnp.dot(x_ref[...], w1_ref[...],
                         preferred_element_type=jnp.float32)


def _main_body(h0_ref, b1_ref, w2_ref, b2_ref, a_ref, o_ref):
    acc = jnp.dot(a_ref[...], h0_ref[...],
                  preferred_element_type=jnp.float32)
    z = jnp.maximum(acc + b1_ref[...], 0.0)
    o_ref[...] = jnp.dot(z, w2_ref[...],
                         preferred_element_type=jnp.float32) + b2_ref[...]


def kernel(x, a, W1, b1, W2, b2):
    n, f = x.shape
    c = W1.shape[1]
    l = W2.shape[1]

    h0 = pl.pallas_call(
        _h0_body,
        out_shape=jax.ShapeDtypeStruct((n, c), jnp.float32),
    )(x, W1)

    b1r = b1.reshape(1, c)
    b2r = b2.reshape(1, l)

    grid = (n // _BM,)
    out = pl.pallas_call(
        _main_body,
        grid=grid,
        in_specs=[
            pl.BlockSpec((n, c), lambda i: (0, 0)),
            pl.BlockSpec((1, c), lambda i: (0, 0)),
            pl.BlockSpec((c, l), lambda i: (0, 0)),
            pl.BlockSpec((1, l), lambda i: (0, 0)),
            pl.BlockSpec((_BM, n), lambda i: (i, 0)),
        ],
        out_specs=pl.BlockSpec((_BM, l), lambda i: (i, 0)),
        out_shape=jax.ShapeDtypeStruct((n, l), jnp.float32),
        compiler_params=pltpu.CompilerParams(
            dimension_semantics=("parallel",),
        ),
    )(h0, b1r, W2, b2r, a)
    return out


# speedup vs baseline: 1.0305x; 1.0305x over previous
import jax
import jax.numpy as jnp
from jax.experimental import pallas as pl
from jax.experimental.pallas import tpu as pltpu

_BM = 400


def _body(x_ref, w1_ref, b1_ref, w2_ref, b2_ref, a_ref, o_ref, h0_s):
    @pl.when(pl.program_id(0) == 0)
    def _():
        h0_s[...] = jnp.dot(x_ref[...], w1_ref[...],
                            preferred_element_type=jnp.float32)

    acc = jnp.dot(a_ref[...], h0_s[...],
                  preferred_element_type=jnp.float32)
    z = jnp.maximum(acc + b1_ref[...], 0.0)
    o_ref[...] = jnp.dot(z, w2_ref[...],
                         preferred_element_type=jnp.float32) + b2_ref[...]


def kernel(x, a, W1, b1, W2, b2):
    n, f = x.shape
    c = W1.shape[1]
    l = W2.shape[1]

    b1r = b1.reshape(1, c)
    b2r = b2.reshape(1, l)

    grid = (n // _BM,)
    out = pl.pallas_call(
        _body,
        grid=grid,
        in_specs=[
            pl.BlockSpec((n, f), lambda i: (0, 0)),
            pl.BlockSpec((f, c), lambda i: (0, 0)),
            pl.BlockSpec((1, c), lambda i: (0, 0)),
            pl.BlockSpec((c, l), lambda i: (0, 0)),
            pl.BlockSpec((1, l), lambda i: (0, 0)),
            pl.BlockSpec((_BM, n), lambda i: (i, 0)),
        ],
        out_specs=pl.BlockSpec((_BM, l), lambda i: (i, 0)),
        out_shape=jax.ShapeDtypeStruct((n, l), jnp.float32),
        scratch_shapes=[pltpu.VMEM((n, c), jnp.float32)],
        compiler_params=pltpu.CompilerParams(
            dimension_semantics=("arbitrary",),
        ),
    )(x, W1, b1r, W2, b2r, a)
    return out
